# baseline (speedup 1.0000x reference)
import functools

import jax
import jax.numpy as jnp
from jax import lax
from jax.experimental import pallas as pl
from jax.experimental.pallas import tpu as pltpu
from jax.experimental.pallas import tpu_sc as plsc

B = 16384
L = 50
DIM = 16
NC, NS, LANES = 2, 16, 16
NW = NC * NS
UPW = B // NW
UG = 64
NG = UPW // UG
IPG = UG * L
CHUNK = 128
NCH = IPG // CHUNK
IDX_ROWS_PER_W = (B * L) // CHUNK // NW
UCHUNK = 64
UIDX_CH = UPW // UCHUNK


def _gmf_body(uidx_hbm, iidx_hbm, utab_hbm, itab_hbm, w_hbm, b_hbm, lu_hbm,
              out_hbm, uidx_v, urows_v, uprime_v, iidx_v, irows_v, out_v,
              wb_v, lu_v, sem):
    wid = lax.axis_index("s") * NC + lax.axis_index("c")
    ubase = wid * UPW

    pltpu.sync_copy(w_hbm, wb_v.at[0])
    pltpu.sync_copy(b_hbm, wb_v.at[1])

    pltpu.sync_copy(uidx_hbm.at[pl.ds(wid * UIDX_CH, UIDX_CH)], uidx_v)
    ucopies = [
        pltpu.async_copy(utab_hbm.at[uidx_v.at[c]],
                         urows_v.at[pl.ds(c * UCHUNK, UCHUNK)], sem)
        for c in range(UIDX_CH)
    ]
    for c in ucopies:
        c.wait()

    pltpu.sync_copy(iidx_hbm.at[pl.ds(wid * IDX_ROWS_PER_W, IDX_ROWS_PER_W)],
                    iidx_v)

    wvec = wb_v[0]

    def _fold(u, _):
        uprime_v[u] = urows_v[u] * wvec
        return _

    lax.fori_loop(0, UPW, _fold, None)

    bvec = wb_v[1]
    iota = lax.iota(jnp.int32, LANES)
    pltpu.sync_copy(lu_hbm, lu_v)

    for g in range(NG):
        icopies = [
            pltpu.async_copy(itab_hbm.at[iidx_v.at[g * NCH + c]],
                             irows_v.at[pl.ds(c * CHUNK, CHUNK)], sem)
            for c in range(NCH)
        ]
        for c in icopies:
            c.wait()

        gu = jnp.int32(g * UG)

        def _tile(t, p):
            lug = lu_v[pl.ds(t * LANES, LANES)] + gu
            acc = jnp.zeros((LANES,), jnp.float32)
            for d in range(DIM):
                dfull = jnp.full((LANES,), d, jnp.int32)
                icol = plsc.load_gather(irows_v, [p, dfull])
                ucol = plsc.load_gather(uprime_v, [lug, dfull])
                acc = acc + icol * ucol
            out_v[pl.ds(t * LANES, LANES)] = acc + bvec
            return p + LANES

        lax.fori_loop(0, IPG // LANES, _tile, iota)

        out0 = ubase * L + g * IPG
        pltpu.sync_copy(out_v, out_hbm.at[pl.ds(out0, IPG)])


@jax.jit
def _gmf_call(uidx2, iidx2, user_table, item_table, w16, b16, lu_all):
    mesh = plsc.VectorSubcoreMesh(core_axis_name="c", subcore_axis_name="s",
                                  num_cores=NC, num_subcores=NS)
    k = pl.kernel(
        _gmf_body,
        out_type=jax.ShapeDtypeStruct((B * L,), jnp.float32),
        mesh=mesh,
        scratch_types=[
            pltpu.VMEM((UIDX_CH, UCHUNK), jnp.int32),
            pltpu.VMEM((UPW, DIM), jnp.float32),
            pltpu.VMEM((UPW, DIM), jnp.float32),
            pltpu.VMEM((IDX_ROWS_PER_W, CHUNK), jnp.int32),
            pltpu.VMEM((IPG, DIM), jnp.float32),
            pltpu.VMEM((IPG,), jnp.float32),
            pltpu.VMEM((2, DIM), jnp.float32),
            pltpu.VMEM((IPG,), jnp.int32),
            pltpu.SemaphoreType.DMA,
        ],
        compiler_params=pltpu.CompilerParams(use_tc_tiling_on_sc=False,
                                             needs_layout_passes=False),
    )
    return k(uidx2, iidx2, user_table, item_table, w16, b16, lu_all)


def kernel(user_indices, item_indices, user_table, item_table, W, b):
    uidx2 = user_indices.reshape(B // UCHUNK, UCHUNK)
    iidx2 = item_indices.reshape((B * L) // CHUNK, CHUNK)
    w16 = W.reshape(DIM)
    b16 = jnp.broadcast_to(b, (DIM,))
    lu_all = jnp.arange(IPG, dtype=jnp.int32) // L
    out = _gmf_call(uidx2, iidx2, user_table, item_table, w16, b16, lu_all)
    return out.reshape(B, L, 1)


# speedup vs baseline: 1.8008x; 1.8008x over previous
import functools

import jax
import jax.numpy as jnp
from jax import lax
from jax.experimental import pallas as pl
from jax.experimental.pallas import tpu as pltpu
from jax.experimental.pallas import tpu_sc as plsc

B = 16384
L = 50
DIM = 16
NV = 1000000
NC, NS, LANES = 2, 16, 16
NW = NC * NS
UPW = B // NW

TB = 192
NBUF = 4
TRIPS = 164
RPW = TB * TRIPS
RPAD = RPW * NW
PADW = RPAD + 1024

UB = 64
NUB = UPW // UB
IPB = UB * L


def _c1_body(itabT_hbm, itab_rm_hbm, tblk_v, trow_v, rsem, wsem):
    wid = lax.axis_index("s") * NC + lax.axis_index("c")
    iota = lax.iota(jnp.int32, LANES)
    base = wid * RPW

    def _read(col0, b, sem):
        return pltpu.async_copy(itabT_hbm.at[:, pl.ds(col0, TB)],
                                tblk_v.at[pl.ds(b * DIM, DIM)], sem)

    def _transpose(b):
        @plsc.parallel_loop(0, TB, unroll=8,
                            carry=jnp.zeros((LANES,), jnp.int32))
        def _pl(r, crloc):
            row = plsc.load_gather(tblk_v, [b * DIM + iota, crloc])
            trow_v[b * TB + r] = row
            return crloc + 1

    for b in range(NBUF):
        _read(base + b * TB, b, rsem)

    def _outer(j4, _):
        j0 = j4 * NBUF
        for b in range(NBUF):
            col0 = base + (j0 + b) * TB
            pltpu.make_async_copy(itabT_hbm.at[:, pl.ds(col0, TB)],
                                  tblk_v.at[pl.ds(b * DIM, DIM)], rsem).wait()

            @pl.when(j4 > 0)
            def _():
                pltpu.make_async_copy(
                    trow_v.at[pl.ds(b * TB, TB)],
                    itab_rm_hbm.at[pl.ds(col0, TB)], wsem).wait()

            _transpose(b)
            pltpu.async_copy(trow_v.at[pl.ds(b * TB, TB)],
                             itab_rm_hbm.at[pl.ds(col0, TB)], wsem)
            _read(col0 + NBUF * TB, b, rsem)
        return _

    lax.fori_loop(0, TRIPS // NBUF, _outer, None)

    for b in range(NBUF):
        pltpu.make_async_copy(itabT_hbm.at[:, pl.ds(base, TB)],
                              tblk_v.at[pl.ds(b * DIM, DIM)], rsem).wait()
        pltpu.make_async_copy(trow_v.at[pl.ds(b * TB, TB)],
                              itab_rm_hbm.at[pl.ds(base, TB)], wsem).wait()


def _c2_body(uidx_hbm, utabT_hbm, iidxT_hbm, itab_rm_hbm, w_hbm, b_hbm,
             out_hbm, uidx_v, upT_v, iidx_v, irows_v, out_v, wb_v,
             sem, gsem, osem):
    wid = lax.axis_index("s") * NC + lax.axis_index("c")
    iota = lax.iota(jnp.int32, LANES)

    pltpu.sync_copy(w_hbm, wb_v.at[0])
    pltpu.sync_copy(b_hbm, wb_v.at[1])

    pltpu.sync_copy(uidx_hbm.at[pl.ds(wid * 4, 4)], uidx_v)
    copies = [
        pltpu.async_copy(utabT_hbm.at[d].at[uidx_v.at[c]],
                         upT_v.at[d, pl.ds(c * 128, 128)], sem)
        for d in range(DIM) for c in range(4)
    ]
    for cp in copies:
        cp.wait()

    wvec = wb_v[0]
    bvec = wb_v[1]
    wb_bc = [jnp.take(wvec, jnp.full((LANES,), d, jnp.int32)) for d in range(DIM)]

    def _issue(ub, slot):
        u0 = wid * UPW + ub * UB
        pltpu.sync_copy(iidxT_hbm.at[:, pl.ds(u0, UB)],
                        iidx_v.at[pl.ds(slot * L, L)])
        for l in range(L):
            pltpu.async_copy(itab_rm_hbm.at[iidx_v.at[slot * L + l]],
                             irows_v.at[pl.ds(slot * IPB + l * UB, UB)], gsem)

    _issue(0, 0)

    def _ub_step(ub, b):
        slot = b
        pltpu.make_async_copy(itab_rm_hbm.at[pl.ds(0, IPB)],
                              irows_v.at[pl.ds(slot * IPB, IPB)], gsem).wait()

        @pl.when(ub + 1 < NUB)
        def _():
            _issue(ub + 1, 1 - slot)

        @pl.when(ub >= 2)
        def _():
            pltpu.make_async_copy(out_v.at[pl.ds(slot * IPB, IPB)],
                                  out_hbm.at[pl.ds(0, IPB)], osem).wait()

        for g in range(UB // LANES):
            ucolw = [upT_v[d, pl.ds(ub * UB + g * LANES, LANES)] * wb_bc[d]
                     for d in range(DIM)]
            row0 = slot * IPB + g * LANES + iota
            pos0 = slot * IPB + (g * LANES + iota) * L

            def _l(l2, carry):
                rowv, posv = carry
                for k in range(2):
                    acc = jnp.zeros((LANES,), jnp.float32)
                    for d in range(DIM):
                        dfull = jnp.full((LANES,), d, jnp.int32)
                        icol = plsc.load_gather(irows_v, [rowv, dfull])
                        acc = acc + icol * ucolw[d]
                    plsc.store_scatter(out_v, [posv], acc + bvec)
                    rowv = rowv + UB
                    posv = posv + 1
                return (rowv, posv)

            lax.fori_loop(0, L // 2, _l, (row0, pos0))

        pltpu.async_copy(out_v.at[pl.ds(slot * IPB, IPB)],
                         out_hbm.at[pl.ds((wid * UPW + ub * UB) * L, IPB)],
                         osem)
        return 1 - b

    lax.fori_loop(0, NUB, _ub_step, 0)

    for s in range(2):
        pltpu.make_async_copy(out_v.at[pl.ds(s * IPB, IPB)],
                              out_hbm.at[pl.ds(0, IPB)], osem).wait()


@jax.jit
def _gmf_call(uidx2, iidxT, utabT, itabT, w16, b16):
    mesh = plsc.VectorSubcoreMesh(core_axis_name="c", subcore_axis_name="s",
                                  num_cores=NC, num_subcores=NS)
    cp = pltpu.CompilerParams(use_tc_tiling_on_sc=False,
                              needs_layout_passes=False)

    c1 = pl.kernel(
        _c1_body,
        out_type=jax.ShapeDtypeStruct((RPAD, DIM), jnp.float32),
        mesh=mesh,
        scratch_types=[
            pltpu.VMEM((NBUF * DIM, TB), jnp.float32),
            pltpu.VMEM((NBUF * TB, DIM), jnp.float32),
            pltpu.SemaphoreType.DMA,
            pltpu.SemaphoreType.DMA,
        ],
        compiler_params=cp,
    )
    itab_rm = c1(itabT)

    c2 = pl.kernel(
        _c2_body,
        out_type=jax.ShapeDtypeStruct((B * L,), jnp.float32),
        mesh=mesh,
        scratch_types=[
            pltpu.VMEM((4, 128), jnp.int32),
            pltpu.VMEM((DIM, UPW), jnp.float32),
            pltpu.VMEM((2 * L, UB), jnp.int32),
            pltpu.VMEM((2 * IPB, DIM), jnp.float32),
            pltpu.VMEM((2 * IPB,), jnp.float32),
            pltpu.VMEM((2, DIM), jnp.float32),
            pltpu.SemaphoreType.DMA,
            pltpu.SemaphoreType.DMA,
            pltpu.SemaphoreType.DMA,
        ],
        compiler_params=cp,
    )
    return c2(uidx2, utabT, iidxT, itab_rm, w16, b16)


def kernel(user_indices, item_indices, user_table, item_table, W, b):
    uidx2 = user_indices.reshape(B // 128, 128)
    iidxT = item_indices.T
    utabT = jnp.pad(user_table.T, ((0, 0), (0, PADW - NV)))
    itabT = jnp.pad(item_table.T, ((0, 0), (0, PADW - NV)))
    w16 = W.reshape(DIM)
    b16 = jnp.broadcast_to(b, (DIM,))
    out = _gmf_call(uidx2, iidxT, utabT, itabT, w16, b16)
    return out.reshape(B, L, 1)
